# Initial kernel scaffold; baseline (speedup 1.0000x reference)
import jax
import jax.numpy as jnp
from jax.experimental import pallas as pl


def kernel(x, edge_attr, edge_index, WQ, WK, WE, WV, WG, bG, WO, bO, ln1_g, ln1_b, W1, b1, W2, b2, ln2_g, ln2_b):
    raise NotImplementedError("write your pallas kernel here")


# speedup vs baseline: 28.6849x; 28.6849x over previous
import jax
import jax.numpy as jnp
from jax import lax
from jax.experimental import pallas as pl
from jax.experimental.pallas import tpu as pltpu
from jax.experimental.pallas import tpu_sc as plsc

N = 10000
E = 320000
D = 128
H = 8
DH = 16

EW = 80
NWIN = 125
EPW = EW * NWIN
NDR = 640


def _proj_body(x_ref, w_ref, k_ref, v_ref, q_ref, g_ref):
    acc = jnp.dot(x_ref[...], w_ref[...], preferred_element_type=jnp.float32)
    k_ref[...] = acc[:, 0:128]
    v_ref[...] = acc[:, 128:256]
    q_ref[...] = acc[:, 256:384]
    g_ref[...] = acc[:, 384:512]


def _node_proj(x, wcat_t):
    blk = 1000
    return pl.pallas_call(
        _proj_body,
        grid=(N // blk,),
        in_specs=[
            pl.BlockSpec((blk, D), lambda i: (i, 0)),
            pl.BlockSpec((D, 512), lambda i: (0, 0)),
        ],
        out_specs=[
            pl.BlockSpec((blk, D), lambda i: (i, 0)),
            pl.BlockSpec((blk, D), lambda i: (i, 0)),
            pl.BlockSpec((blk, D), lambda i: (i, 0)),
            pl.BlockSpec((blk, D), lambda i: (i, 0)),
        ],
        out_shape=[
            jax.ShapeDtypeStruct((N, D), jnp.float32),
            jax.ShapeDtypeStruct((N, D), jnp.float32),
            jax.ShapeDtypeStruct((N, D), jnp.float32),
            jax.ShapeDtypeStruct((N, D), jnp.float32),
        ],
    )(x, wcat_t)


def _eh_body(ea_ref, w_ref, out_ref):
    out_ref[...] = jnp.dot(ea_ref[...], w_ref[...],
                           preferred_element_type=jnp.float32)


def _edge_proj(edge_attr, we_t):
    blk = 2000
    return pl.pallas_call(
        _eh_body,
        grid=(E // blk,),
        in_specs=[
            pl.BlockSpec((blk, D), lambda i: (i, 0)),
            pl.BlockSpec((D, D), lambda i: (0, 0)),
        ],
        out_specs=pl.BlockSpec((blk, D), lambda i: (i, 0)),
        out_shape=jax.ShapeDtypeStruct((E, D), jnp.float32),
    )(edge_attr, we_t)


_TAKE_DNUMS = lax.GatherDimensionNumbers(
    offset_dims=(), collapsed_slice_dims=(0,), start_index_map=(0,))


def _take16(x, idx):
    return lax.gather(x, idx.reshape(16, 1), _TAKE_DNUMS, (1,),
                      mode=lax.GatherScatterMode.PROMISE_IN_BOUNDS)


def _edge_body(k_hbm, v_hbm, q_hbm, eh_hbm, src_hbm, dst_hbm,
               msg_out, den_out,
               acc_sh, den_sh,
               src_v, dst_v, drow_v, g_v, q_v, p_v, den_stage, sem):
    cid = lax.axis_index("c")
    sid = lax.axis_index("s")
    wid = sid * 2 + cid
    ebase = wid * EPW

    zero16 = jnp.zeros((16,), jnp.float32)
    iota16 = lax.iota(jnp.int32, 16)

    def _zrow(i, carry):
        for c8 in range(8):
            den_stage[i, pl.ds(c8 * 16, 16)] = zero16
        return carry
    lax.fori_loop(0, EW, _zrow, 0)

    NB = N // EW
    def _zacc(k, carry):
        b = sid + k * 16

        @pl.when(b < NB)
        def _():
            pltpu.sync_copy(den_stage, acc_sh.at[pl.ds(b * EW, EW)])
        return carry
    lax.fori_loop(0, (NB + 15) // 16, _zacc, 0)
    pltpu.sync_copy(den_stage.at[pl.ds(0, NDR // 16)],
                    den_sh.at[pl.ds(sid * (NDR // 16), NDR // 16)])
    plsc.subcore_barrier()

    def _window(w, carry):
        est = ebase + w * EW
        pltpu.sync_copy(src_hbm.at[pl.ds(est, EW)], src_v)
        pltpu.sync_copy(dst_hbm.at[pl.ds(est, EW)], dst_v)
        pltpu.async_copy(k_hbm.at[src_v], g_v, sem).wait()
        pltpu.async_copy(q_hbm.at[dst_v], q_v, sem).wait()
        pltpu.sync_copy(eh_hbm.at[pl.ds(est, EW)], p_v)

        def _group(g, gcarry):
            dvec = dst_v[pl.ds(g * 16, 16)]
            subv = dvec & 15
            offv = (subv >> 1) << 4
            oddv = (subv & 1) << 3
            drow_v[pl.ds(g * 16, 16)] = dvec >> 4
            for e16 in range(16):
                e = g * 16 + e16
                pbs = []
                sv = zero16
                for h in range(H):
                    k = g_v[e, pl.ds(h * DH, DH)]
                    q = q_v[e, pl.ds(h * DH, DH)]
                    eh = p_v[e, pl.ds(h * DH, DH)]
                    t = k * q * eh
                    t = t + _take16(t, iota16 ^ 8)
                    t = t + _take16(t, iota16 ^ 4)
                    t = t + _take16(t, iota16 ^ 2)
                    t = t + _take16(t, iota16 ^ 1)
                    pbs.append(jnp.exp(t))
                    p_v[e, pl.ds(h * DH, DH)] = pbs[h]
                    sv = jnp.where(iota16 == h, pbs[h], sv)
                pe = sv
                odd8 = oddv[e16]
                shifted = _take16(pe, (iota16 - odd8) & 15)
                lanemask = (iota16 >= odd8) & (iota16 < odd8 + 8)
                den_stage[e, pl.ds(offv[e16], DH)] = jnp.where(
                    lanemask, shifted, 0.0)
            return gcarry
        lax.fori_loop(0, EW // 16, _group, 0)

        pltpu.async_copy(v_hbm.at[src_v], g_v, sem).wait()

        def _scale(e, scarry):
            for c8 in range(8):
                sl = pl.ds(c8 * 16, 16)
                g_v[e, sl] = g_v[e, sl] * p_v[e, sl]
            return scarry
        lax.fori_loop(0, EW, _scale, 0)

        pltpu.sync_copy(g_v, acc_sh.at[dst_v], add=True)
        pltpu.sync_copy(den_stage, den_sh.at[drow_v], add=True)

        def _rez(g, gcarry):
            offv = ((dst_v[pl.ds(g * 16, 16)] & 15) >> 1) << 4
            for e16 in range(16):
                den_stage[g * 16 + e16, pl.ds(offv[e16], DH)] = zero16
            return gcarry
        lax.fori_loop(0, EW // 16, _rez, 0)
        return carry
    lax.fori_loop(0, NWIN, _window, 0)

    plsc.subcore_barrier()
    NB2 = N // EW

    def _wout(k, carry):
        b = sid + k * 16

        @pl.when(b < NB2)
        def _():
            pltpu.sync_copy(acc_sh.at[pl.ds(b * EW, EW)],
                            msg_out.at[cid, pl.ds(b * EW, EW)])
        return carry
    lax.fori_loop(0, (NB2 + 15) // 16, _wout, 0)
    pltpu.sync_copy(den_sh.at[pl.ds(sid * (NDR // 16), NDR // 16)],
                    den_out.at[cid, pl.ds(sid * (NDR // 16), NDR // 16)])


def _edge_stage(k, v, q, ehm, src, dst):
    mesh = plsc.VectorSubcoreMesh(core_axis_name="c", subcore_axis_name="s")
    fn = pl.kernel(
        _edge_body,
        out_type=[
            jax.ShapeDtypeStruct((2, N, D), jnp.float32),
            jax.ShapeDtypeStruct((2, NDR, D), jnp.float32),
        ],
        mesh=mesh,
        scratch_types=[
            pltpu.VMEM_SHARED((N, D), jnp.float32),
            pltpu.VMEM_SHARED((NDR, D), jnp.float32),
            pltpu.VMEM((EW,), jnp.int32),
            pltpu.VMEM((EW,), jnp.int32),
            pltpu.VMEM((EW,), jnp.int32),
            pltpu.VMEM((EW, D), jnp.float32),
            pltpu.VMEM((EW, D), jnp.float32),
            pltpu.VMEM((EW, D), jnp.float32),
            pltpu.VMEM((EW, D), jnp.float32),
            pltpu.SemaphoreType.DMA,
        ],
    )
    return fn(k, v, q, ehm, src, dst)


def _epi_body(x_ref, acc_ref, den_ref, gpre_ref, exp_ref, bg_ref, wo_ref,
              bo_ref, ln1g_ref, ln1b_ref, w1_ref, b1_ref, w2_ref, b2_ref,
              ln2g_ref, ln2b_ref, out_ref):
    num = acc_ref[0] + acc_ref[1]
    den = den_ref[0] + den_ref[1]
    den128 = jnp.dot(den, exp_ref[...], preferred_element_type=jnp.float32)
    wv = num / (den128 + 1e-16)
    g = jax.nn.sigmoid(gpre_ref[...] + bg_ref[...])
    h = wv * g
    h = jnp.dot(h, wo_ref[...], preferred_element_type=jnp.float32) + bo_ref[...]
    h = x_ref[...] + h
    mu = jnp.mean(h, axis=-1, keepdims=True)
    var = jnp.mean((h - mu) ** 2, axis=-1, keepdims=True)
    h = (h - mu) / jnp.sqrt(var + 1e-5) * ln1g_ref[...] + ln1b_ref[...]
    ff = jnp.maximum(
        jnp.dot(h, w1_ref[...], preferred_element_type=jnp.float32)
        + b1_ref[...], 0.0)
    ff = jnp.dot(ff, w2_ref[...], preferred_element_type=jnp.float32) + b2_ref[...]
    h2 = h + ff
    mu2 = jnp.mean(h2, axis=-1, keepdims=True)
    var2 = jnp.mean((h2 - mu2) ** 2, axis=-1, keepdims=True)
    out_ref[...] = ((h2 - mu2) / jnp.sqrt(var2 + 1e-5) * ln2g_ref[...]
                    + ln2b_ref[...])


def _epilogue(x, acc, den2, gpre, expand, bG, wo_t, bO, ln1_g, ln1_b,
              w1_t, b1, w2_t, b2, ln2_g, ln2_b):
    blk = 1000
    row = lambda i: (i, 0)
    cst = lambda i: (0, 0)
    return pl.pallas_call(
        _epi_body,
        grid=(N // blk,),
        in_specs=[
            pl.BlockSpec((blk, D), row),
            pl.BlockSpec((2, blk, D), lambda i: (0, i, 0)),
            pl.BlockSpec((2, blk, 8), lambda i: (0, i, 0)),
            pl.BlockSpec((blk, D), row),
            pl.BlockSpec((8, D), cst),
            pl.BlockSpec((1, D), cst),
            pl.BlockSpec((D, D), cst),
            pl.BlockSpec((1, D), cst),
            pl.BlockSpec((1, D), cst),
            pl.BlockSpec((1, D), cst),
            pl.BlockSpec((D, 2 * D), cst),
            pl.BlockSpec((1, 2 * D), cst),
            pl.BlockSpec((2 * D, D), cst),
            pl.BlockSpec((1, D), cst),
            pl.BlockSpec((1, D), cst),
            pl.BlockSpec((1, D), cst),
        ],
        out_specs=pl.BlockSpec((blk, D), row),
        out_shape=jax.ShapeDtypeStruct((N, D), jnp.float32),
    )(x, acc, den2, gpre, expand, bG, wo_t, bO, ln1_g, ln1_b,
      w1_t, b1, w2_t, b2, ln2_g, ln2_b)


def kernel(x, edge_attr, edge_index, WQ, WK, WE, WV, WG, bG, WO, bO,
           ln1_g, ln1_b, W1, b1, W2, b2, ln2_g, ln2_b):
    wcat_t = jnp.concatenate([WK, WV, WQ * 0.25, WG], axis=0).T
    k, v, q, gpre = _node_proj(x, wcat_t)
    ehm = _edge_proj(edge_attr, WE.T)
    src = edge_index[0]
    dst = edge_index[1]
    acc, den = _edge_stage(k, v, q, ehm, src, dst)
    den2 = den.reshape(2, NDR * 16, 8)[:, :N, :]
    expand = jnp.repeat(jnp.eye(8, dtype=jnp.float32), DH, axis=1)
    out = _epilogue(
        x, acc, den2, gpre, expand,
        bG.reshape(1, D), WO.T, bO.reshape(1, D),
        ln1_g.reshape(1, D), ln1_b.reshape(1, D),
        W1.T, b1.reshape(1, 2 * D), W2.T, b2.reshape(1, D),
        ln2_g.reshape(1, D), ln2_b.reshape(1, D),
    )
    return out
